# baseline (speedup 1.0000x reference)
import jax
import jax.numpy as jnp
from jax.experimental import pallas as pl
from jax.experimental.pallas import tpu as pltpu

_P = 3
_NBUF = 6


def _argmax_manual(x_hbm, o_ref, vbuf, val_ref, idx_ref, sems):
    t = pl.program_id(0)
    g = pl.num_programs(0)

    def copy(c, slot):
        return pltpu.make_async_copy(
            x_hbm.at[pl.ds(c * _P, _P)],
            vbuf.at[slot],
            sems.at[slot],
        )

    @pl.when(t == 0)
    def _():
        val_ref[...] = jnp.full(val_ref.shape, -jnp.inf, jnp.float32)
        idx_ref[...] = jnp.zeros(idx_ref.shape, jnp.int32)
        for d in range(_NBUF - 1):
            copy(d, d).start()

    @pl.when(t + _NBUF - 1 < g)
    def _():
        c = t + _NBUF - 1
        copy(c, c % _NBUF).start()

    slot = t % _NBUF
    copy(t, slot).wait()

    base = t * _P
    s = val_ref.shape[1]
    tl = 256
    for c in range(s // tl):
        sl = pl.ds(c * tl, tl)
        val = val_ref[:, sl]
        idx = idx_ref[:, sl]
        for p in range(_P):
            xp = vbuf[slot, p, :, sl]
            better = xp > val
            val = jnp.where(better, xp, val)
            idx = jnp.where(better, base + p, idx)
        val_ref[:, sl] = val
        idx_ref[:, sl] = idx

    @pl.when(t == g - 1)
    def _():
        o_ref[...] = idx_ref[...]


def kernel(input_0):
    b, s, n = input_0.shape
    assert n % _P == 0
    xt = jnp.transpose(input_0, (2, 0, 1))
    out = pl.pallas_call(
        _argmax_manual,
        grid=(n // _P,),
        in_specs=[pl.BlockSpec(memory_space=pltpu.MemorySpace.HBM)],
        out_specs=pl.BlockSpec((b, s), lambda t: (0, 0)),
        out_shape=jax.ShapeDtypeStruct((b, s), jnp.int32),
        scratch_shapes=[
            pltpu.VMEM((_NBUF, _P, b, s), jnp.float32),
            pltpu.VMEM((b, s), jnp.float32),
            pltpu.VMEM((b, s), jnp.int32),
            pltpu.SemaphoreType.DMA((_NBUF,)),
        ],
        compiler_params=pltpu.CompilerParams(
            dimension_semantics=("arbitrary",)
        ),
    )(xt)
    return out.astype(jnp.int64)


# speedup vs baseline: 1.1262x; 1.1262x over previous
import jax
import jax.numpy as jnp
from jax.experimental import pallas as pl
from jax.experimental.pallas import tpu as pltpu

_P = 9


def _argmax_planes(x_hbm, o_ref, vbuf, val_ref, idx_ref, sems):
    t = pl.program_id(0)
    g = pl.num_programs(0)

    def copy(c, p, slot):
        return pltpu.make_async_copy(
            x_hbm.at[c * _P + p],
            vbuf.at[slot, p],
            sems.at[slot],
        )

    @pl.when(t == 0)
    def _():
        val_ref[...] = jnp.full(val_ref.shape, -jnp.inf, jnp.float32)
        idx_ref[...] = jnp.zeros(idx_ref.shape, jnp.int32)
        for p in range(_P):
            copy(0, p, 0).start()

    @pl.when(t + 1 < g)
    def _():
        for p in range(_P):
            copy(t + 1, p, (t + 1) % 2).start()

    slot = t % 2
    for p in range(_P):
        copy(t, p, slot).wait()

    base = t * _P
    s = val_ref.shape[1]
    tl = 256
    for c in range(s // tl):
        sl = pl.ds(c * tl, tl)
        val = val_ref[:, sl]
        idx = idx_ref[:, sl]
        for p in range(_P):
            xp = vbuf[slot, p, :, sl]
            better = xp > val
            val = jnp.where(better, xp, val)
            idx = jnp.where(better, base + p, idx)
        val_ref[:, sl] = val
        idx_ref[:, sl] = idx

    @pl.when(t == g - 1)
    def _():
        o_ref[...] = idx_ref[...]


def kernel(input_0):
    b, s, n = input_0.shape
    assert n % _P == 0
    xt = jnp.transpose(input_0, (2, 0, 1))
    out = pl.pallas_call(
        _argmax_planes,
        grid=(n // _P,),
        in_specs=[pl.BlockSpec(memory_space=pltpu.MemorySpace.HBM)],
        out_specs=pl.BlockSpec((b, s), lambda t: (0, 0)),
        out_shape=jax.ShapeDtypeStruct((b, s), jnp.int32),
        scratch_shapes=[
            pltpu.VMEM((2, _P, b, s), jnp.float32),
            pltpu.VMEM((b, s), jnp.float32),
            pltpu.VMEM((b, s), jnp.int32),
            pltpu.SemaphoreType.DMA((2,)),
        ],
        compiler_params=pltpu.CompilerParams(
            dimension_semantics=("arbitrary",)
        ),
    )(xt)
    return out.astype(jnp.int64)


# speedup vs baseline: 1.1719x; 1.0406x over previous
import jax
import jax.numpy as jnp
from jax.experimental import pallas as pl
from jax.experimental.pallas import tpu as pltpu

_P = 13


def _argmax_planes(x_ref, o_ref, val_ref, idx_ref):
    k = pl.program_id(0)
    nk = pl.num_programs(0)

    @pl.when(k == 0)
    def _():
        val_ref[...] = jnp.full(val_ref.shape, -jnp.inf, jnp.float32)
        idx_ref[...] = jnp.zeros(idx_ref.shape, jnp.int32)

    base = k * _P
    s = x_ref.shape[2]
    tl = 256
    for c in range(s // tl):
        sl = pl.ds(c * tl, tl)
        val = val_ref[:, sl]
        idx = idx_ref[:, sl]
        for p in range(_P):
            xp = x_ref[p, :, sl]
            better = xp > val
            val = jnp.where(better, xp, val)
            idx = jnp.where(better, base + p, idx)
        val_ref[:, sl] = val
        idx_ref[:, sl] = idx

    @pl.when(k == nk - 1)
    def _():
        o_ref[...] = idx_ref[...]


def kernel(input_0):
    b, s, n = input_0.shape
    assert n % _P == 0
    xt = jnp.transpose(input_0, (2, 0, 1))
    out = pl.pallas_call(
        _argmax_planes,
        grid=(n // _P,),
        in_specs=[pl.BlockSpec((_P, b, s), lambda k: (k, 0, 0))],
        out_specs=pl.BlockSpec((b, s), lambda k: (0, 0)),
        out_shape=jax.ShapeDtypeStruct((b, s), jnp.int32),
        scratch_shapes=[
            pltpu.VMEM((b, s), jnp.float32),
            pltpu.VMEM((b, s), jnp.int32),
        ],
        compiler_params=pltpu.CompilerParams(
            dimension_semantics=("arbitrary",),
            vmem_limit_bytes=66 * 1024 * 1024,
        ),
    )(xt)
    return out.astype(jnp.int64)
